# Initial kernel scaffold; baseline (speedup 1.0000x reference)
import jax
import jax.numpy as jnp
from jax.experimental import pallas as pl


def kernel(grid_features, g2m_features, m2g_features, mesh_static_features, m2m_features, params, m2m_edge_index, g2m_edge_index, m2g_edge_index):
    raise NotImplementedError("write your pallas kernel here")


# speedup vs baseline: 2.2124x; 2.2124x over previous
import functools

import jax
import jax.numpy as jnp
from jax import lax
from jax.experimental import pallas as pl
from jax.experimental.pallas import tpu as pltpu
from jax.experimental.pallas import tpu_sc as plsc

H = 64
_NC = 2
_NS = 16
_NW = _NC * _NS
_CH = 128
_F32 = jnp.float32


def _silu(x):
    return x * jax.nn.sigmoid(x)


def _ln(x, g, b):
    mu = jnp.mean(x, axis=-1, keepdims=True)
    var = jnp.mean((x - mu) ** 2, axis=-1, keepdims=True)
    return (x - mu) / jnp.sqrt(var + 1e-5) * g + b


def _full(a):
    nd = a.ndim
    return pl.BlockSpec(a.shape, lambda i, _nd=nd: (0,) * _nd)


def _pcall(body, row_args, full_args, n, dout, block):
    in_specs = [
        pl.BlockSpec((block,) + a.shape[1:], lambda i, _nd=a.ndim: (i,) + (0,) * (_nd - 1))
        for a in row_args
    ] + [_full(a) for a in full_args]
    return pl.pallas_call(
        body,
        grid=(n // block,),
        in_specs=in_specs,
        out_specs=pl.BlockSpec((block, dout), lambda i: (i, 0)),
        out_shape=jax.ShapeDtypeStruct((n, dout), _F32),
    )(*row_args, *full_args)


def _mlp2(x, p, block=2000):
    w0, w1 = p["W"]
    b0, b1 = [b.reshape(1, -1) for b in p["b"]]
    dout = w1.shape[1]
    ln = p.get("ln")
    if ln is not None:
        g, be = [t.reshape(1, -1) for t in ln]

        def body(x_r, w0_r, b0_r, w1_r, b1_r, g_r, be_r, o_r):
            h = _silu(jnp.dot(x_r[...], w0_r[...], preferred_element_type=_F32) + b0_r[...])
            y = jnp.dot(h, w1_r[...], preferred_element_type=_F32) + b1_r[...]
            o_r[...] = _ln(y, g_r[...], be_r[...])

        return _pcall(body, [x], [w0, b0, w1, b1, g, be], x.shape[0], dout, block)

    def body(x_r, w0_r, b0_r, w1_r, b1_r, o_r):
        h = _silu(jnp.dot(x_r[...], w0_r[...], preferred_element_type=_F32) + b0_r[...])
        o_r[...] = jnp.dot(h, w1_r[...], preferred_element_type=_F32) + b1_r[...]

    return _pcall(body, [x], [w0, b0, w1, b1], x.shape[0], dout, block)


def _linear(x, w, block=2000):
    def body(x_r, w_r, o_r):
        o_r[...] = jnp.dot(x_r[...], w_r[...], preferred_element_type=_F32)

    return _pcall(body, [x], [w], x.shape[0], w.shape[1], block)


def _edge_dense(xe, gs, gr, embp, w1e, b1, w2, b2, g2, be2, block=2000):
    wa, wb = embp["W"]
    ba, bb = [b.reshape(1, -1) for b in embp["b"]]
    ge, bee = [t.reshape(1, -1) for t in embp["ln"]]
    b1 = b1.reshape(1, -1)
    b2 = b2.reshape(1, -1)
    g2 = g2.reshape(1, -1)
    be2 = be2.reshape(1, -1)

    def body(xe_r, gs_r, gr_r, wa_r, ba_r, wb_r, bb_r, ge_r, bee_r,
             w1e_r, b1_r, w2_r, b2_r, g2_r, be2_r, o_r):
        t = _silu(jnp.dot(xe_r[...], wa_r[...], preferred_element_type=_F32) + ba_r[...])
        u = jnp.dot(t, wb_r[...], preferred_element_type=_F32) + bb_r[...]
        v = _ln(u, ge_r[...], bee_r[...])
        w = _silu(jnp.dot(v, w1e_r[...], preferred_element_type=_F32)
                  + gs_r[...] + gr_r[...] + b1_r[...])
        d = jnp.dot(w, w2_r[...], preferred_element_type=_F32) + b2_r[...]
        o_r[...] = _ln(d, g2_r[...], be2_r[...])

    return _pcall(body, [xe, gs, gr],
                  [wa, ba, wb, bb, ge, bee, w1e, b1, w2, b2, g2, be2],
                  xe.shape[0], H, block)


def _node_update(rec, parts, w3r, w3a, b3, w4, b4, g, be, split):
    R = rec.shape[0]
    b3 = b3.reshape(1, -1)
    b4 = b4.reshape(1, -1)
    g = g.reshape(1, -1)
    be = be.reshape(1, -1)
    full = [w3r, w3a, b3, w4, b4, g, be]

    def tail(rec_v, a, w3r_r, w3a_r, b3_r, w4_r, b4_r, g_r, be_r):
        h = _silu(jnp.dot(rec_v, w3r_r[...], preferred_element_type=_F32)
                  + jnp.dot(a, w3a_r[...], preferred_element_type=_F32) + b3_r[...])
        y = jnp.dot(h, w4_r[...], preferred_element_type=_F32) + b4_r[...]
        return rec_v + _ln(y, g_r[...], be_r[...])

    if split:
        block = 1000
        hb = (R // block) // 2

        def body(rec_r, pp_r, *rest):
            o_r = rest[-1]
            o_r[...] = tail(rec_r[...], pp_r[0], *rest[:-1])

        pspec = pl.BlockSpec((1, block, H), lambda i: (i // hb, i % hb, 0))
    else:
        block = 2000

        def body(rec_r, pp_r, *rest):
            o_r = rest[-1]
            o_r[...] = tail(rec_r[...], pp_r[0] + pp_r[1], *rest[:-1])

        pspec = pl.BlockSpec((2, block, H), lambda i: (0, i, 0))

    in_specs = [pl.BlockSpec((block, H), lambda i: (i, 0)), pspec] + [_full(a) for a in full]
    return pl.pallas_call(
        body,
        grid=(R // block,),
        in_specs=in_specs,
        out_specs=pl.BlockSpec((block, H), lambda i: (i, 0)),
        out_shape=jax.ShapeDtypeStruct((R, H), _F32),
    )(rec, parts, *full)


def _sc_gather(stab, rtab, sidx, didx):
    nch = sidx.shape[0]
    E = nch * _CH
    mesh = plsc.VectorSubcoreMesh(core_axis_name="c", subcore_axis_name="s")

    @functools.partial(
        pl.kernel,
        out_type=(jax.ShapeDtypeStruct((E, H), _F32), jax.ShapeDtypeStruct((E, H), _F32)),
        mesh=mesh,
        compiler_params=pltpu.CompilerParams(use_tc_tiling_on_sc=False),
        scratch_types=[
            pltpu.VMEM((_CH,), jnp.int32),
            pltpu.VMEM((_CH,), jnp.int32),
            pltpu.VMEM((_CH, H), _F32),
            pltpu.VMEM((_CH, H), _F32),
            pltpu.SemaphoreType.DMA,
            pltpu.SemaphoreType.DMA,
        ],
    )
    def k(stab_h, rtab_h, sidx_h, didx_h, gs_h, gr_h, siv, div, sbuf, rbuf, s1, s2):
        wid = lax.axis_index("s") * _NC + lax.axis_index("c")
        ntr = (nch - wid + _NW - 1) // _NW

        def body(i, carry):
            j = wid + i * _NW
            pltpu.sync_copy(sidx_h.at[j], siv)
            pltpu.sync_copy(didx_h.at[j], div)
            c1 = pltpu.async_copy(stab_h.at[siv], sbuf, s1)
            c2 = pltpu.async_copy(rtab_h.at[div], rbuf, s2)
            c1.wait()
            c2.wait()
            pltpu.sync_copy(sbuf, gs_h.at[pl.ds(j * _CH, _CH)])
            pltpu.sync_copy(rbuf, gr_h.at[pl.ds(j * _CH, _CH)])
            return carry

        lax.fori_loop(0, ntr, body, 0)

    return k(stab, rtab, sidx, didx)


def _sc_scatter(ed, didx, num_rec, split):
    E = ed.shape[0]
    nch = E // _CH
    if split:
        half = num_rec // 2
        racc = half + 8
    else:
        half = num_rec
        racc = num_rec
    rps = racc // _NS
    zeros = jnp.zeros((racc, H), _F32)
    mesh = plsc.VectorSubcoreMesh(core_axis_name="c", subcore_axis_name="s")

    @functools.partial(
        pl.kernel,
        out_type=jax.ShapeDtypeStruct((2, racc, H), _F32),
        mesh=mesh,
        compiler_params=pltpu.CompilerParams(use_tc_tiling_on_sc=False),
        scratch_types=[
            pltpu.VMEM((_CH,), jnp.int32),
            pltpu.VMEM((_CH, H), _F32),
            pltpu.VMEM_SHARED((racc, H), _F32),
        ],
    )
    def k(ed_h, didx_h, z_h, out_h, idxv, rows, accum):
        c = lax.axis_index("c")
        s = lax.axis_index("s")
        pltpu.sync_copy(z_h.at[pl.ds(s * rps, rps)], accum.at[pl.ds(s * rps, rps)])
        plsc.subcore_barrier()
        if split:
            start = s
            stride = _NS
        else:
            start = s * _NC + c
            stride = _NW
        ntr = (nch - start + stride - 1) // stride

        def body(i, carry):
            j = start + i * stride
            pltpu.sync_copy(didx_h.at[j], idxv)
            pltpu.sync_copy(ed_h.at[pl.ds(j * _CH, _CH)], rows)
            if split:
                base = c * half
                for t in range(_CH // 16):
                    v = idxv[pl.ds(t * 16, 16)]
                    li = v - base
                    ok = (li >= 0) & (li < half)
                    idxv[pl.ds(t * 16, 16)] = jnp.where(ok, li, half)
            pltpu.sync_copy(rows, accum.at[idxv], add=True)
            return carry

        lax.fori_loop(0, ntr, body, 0)
        plsc.subcore_barrier()
        pltpu.sync_copy(accum.at[pl.ds(s * rps, rps)], out_h.at[c, pl.ds(s * rps, rps)])

    return k(ed, didx, zeros)


def _interaction(gnnp, embp, edge_index, send_rep, rec_rep, edge_feats, num_rec, split):
    em = gnnp["edge_mlp"]
    am = gnnp["aggr_mlp"]
    w1 = em["W"][0]
    w1e, w1s, w1r = w1[:H], w1[H:2 * H], w1[2 * H:]
    b1, b2 = em["b"]
    w2 = em["W"][1]
    g2, be2 = em["ln"]
    w3 = am["W"][0]
    w3r, w3a = w3[:H], w3[H:]
    b3, b4 = am["b"]
    w4 = am["W"][1]
    g4, be4 = am["ln"]

    src = edge_index[0].astype(jnp.int32)
    dst = edge_index[1].astype(jnp.int32)
    E = src.shape[0]
    sidx = src.reshape(E // _CH, _CH)
    didx = dst.reshape(E // _CH, _CH)

    sproj = _linear(send_rep, w1s)
    rproj = _linear(rec_rep, w1r)
    gs, gr = _sc_gather(sproj, rproj, sidx, didx)
    ed = _edge_dense(edge_feats, gs, gr, embp, w1e, b1, w2, b2, g2, be2)
    parts = _sc_scatter(ed, didx, num_rec, split)
    return _node_update(rec_rep, parts, w3r, w3a, b3, w4, b4, g4, be4, split)


def kernel(grid_features, g2m_features, m2g_features, mesh_static_features,
           m2m_features, params, m2m_edge_index, g2m_edge_index, m2g_edge_index):
    p = params
    grid_emb = _mlp2(grid_features, p["grid_embedder"])
    mesh_emb = _mlp2(mesh_static_features, p["mesh_embedder"])
    mesh_rep = _interaction(p["g2m_gnn"], p["g2m_embedder"], g2m_edge_index,
                            grid_emb, mesh_emb, g2m_features, 10000, False)
    grid_rep = _mlp2(grid_emb, p["encoding_grid_mlp"])
    mesh_rep = _interaction(p["processor"], p["m2m_embedder"], m2m_edge_index,
                            mesh_rep, mesh_rep, m2m_features, 10000, False)
    grid_rep = _interaction(p["m2g_gnn"], p["m2g_embedder"], m2g_edge_index,
                            mesh_rep, grid_rep, m2g_features, 50000, True)
    return _mlp2(grid_rep, p["output_map"])


# speedup vs baseline: 2.5428x; 1.1493x over previous
import functools

import jax
import jax.numpy as jnp
from jax import lax
from jax.experimental import pallas as pl
from jax.experimental.pallas import tpu as pltpu
from jax.experimental.pallas import tpu_sc as plsc

H = 64
_NC = 2
_NS = 16
_NW = _NC * _NS
_CH = 128
_F32 = jnp.float32


def _silu(x):
    return x * jax.nn.sigmoid(x)


def _ln(x, g, b):
    mu = jnp.mean(x, axis=-1, keepdims=True)
    var = jnp.mean((x - mu) ** 2, axis=-1, keepdims=True)
    return (x - mu) / jnp.sqrt(var + 1e-5) * g + b


def _full(a):
    nd = a.ndim
    return pl.BlockSpec(a.shape, lambda i, _nd=nd: (0,) * _nd)


def _pcall(body, row_args, full_args, n, dout, block):
    in_specs = [
        pl.BlockSpec((block,) + a.shape[1:], lambda i, _nd=a.ndim: (i,) + (0,) * (_nd - 1))
        for a in row_args
    ] + [_full(a) for a in full_args]
    return pl.pallas_call(
        body,
        grid=(n // block,),
        in_specs=in_specs,
        out_specs=pl.BlockSpec((block, dout), lambda i: (i, 0)),
        out_shape=jax.ShapeDtypeStruct((n, dout), _F32),
    )(*row_args, *full_args)


def _mlp2(x, p, block=2000):
    w0, w1 = p["W"]
    b0, b1 = [b.reshape(1, -1) for b in p["b"]]
    dout = w1.shape[1]
    ln = p.get("ln")
    if ln is not None:
        g, be = [t.reshape(1, -1) for t in ln]

        def body(x_r, w0_r, b0_r, w1_r, b1_r, g_r, be_r, o_r):
            h = _silu(jnp.dot(x_r[...], w0_r[...], preferred_element_type=_F32) + b0_r[...])
            y = jnp.dot(h, w1_r[...], preferred_element_type=_F32) + b1_r[...]
            o_r[...] = _ln(y, g_r[...], be_r[...])

        return _pcall(body, [x], [w0, b0, w1, b1, g, be], x.shape[0], dout, block)

    def body(x_r, w0_r, b0_r, w1_r, b1_r, o_r):
        h = _silu(jnp.dot(x_r[...], w0_r[...], preferred_element_type=_F32) + b0_r[...])
        o_r[...] = jnp.dot(h, w1_r[...], preferred_element_type=_F32) + b1_r[...]

    return _pcall(body, [x], [w0, b0, w1, b1], x.shape[0], dout, block)


def _linear(x, w, block=2000):
    def body(x_r, w_r, o_r):
        o_r[...] = jnp.dot(x_r[...], w_r[...], preferred_element_type=_F32)

    return _pcall(body, [x], [w], x.shape[0], w.shape[1], block)


def _edge_dense(xe, gs, gr, embp, w1e, b1, w2, b2, g2, be2, block=2000):
    wa, wb = embp["W"]
    ba, bb = [b.reshape(1, -1) for b in embp["b"]]
    ge, bee = [t.reshape(1, -1) for t in embp["ln"]]
    b1 = b1.reshape(1, -1)
    b2 = b2.reshape(1, -1)
    g2 = g2.reshape(1, -1)
    be2 = be2.reshape(1, -1)

    def body(xe_r, gs_r, gr_r, wa_r, ba_r, wb_r, bb_r, ge_r, bee_r,
             w1e_r, b1_r, w2_r, b2_r, g2_r, be2_r, o_r):
        t = _silu(jnp.dot(xe_r[...], wa_r[...], preferred_element_type=_F32) + ba_r[...])
        u = jnp.dot(t, wb_r[...], preferred_element_type=_F32) + bb_r[...]
        v = _ln(u, ge_r[...], bee_r[...])
        w = _silu(jnp.dot(v, w1e_r[...], preferred_element_type=_F32)
                  + gs_r[...] + gr_r[...] + b1_r[...])
        d = jnp.dot(w, w2_r[...], preferred_element_type=_F32) + b2_r[...]
        o_r[...] = _ln(d, g2_r[...], be2_r[...])

    return _pcall(body, [xe, gs, gr],
                  [wa, ba, wb, bb, ge, bee, w1e, b1, w2, b2, g2, be2],
                  xe.shape[0], H, block)


def _node_update(rec, parts, w3r, w3a, b3, w4, b4, g, be, split):
    R = rec.shape[0]
    b3 = b3.reshape(1, -1)
    b4 = b4.reshape(1, -1)
    g = g.reshape(1, -1)
    be = be.reshape(1, -1)
    full = [w3r, w3a, b3, w4, b4, g, be]

    def tail(rec_v, a, w3r_r, w3a_r, b3_r, w4_r, b4_r, g_r, be_r):
        h = _silu(jnp.dot(rec_v, w3r_r[...], preferred_element_type=_F32)
                  + jnp.dot(a, w3a_r[...], preferred_element_type=_F32) + b3_r[...])
        y = jnp.dot(h, w4_r[...], preferred_element_type=_F32) + b4_r[...]
        return rec_v + _ln(y, g_r[...], be_r[...])

    if split:
        block = 1000
        hb = (R // block) // 2

        def body(rec_r, pp_r, *rest):
            o_r = rest[-1]
            o_r[...] = tail(rec_r[...], pp_r[0], *rest[:-1])

        pspec = pl.BlockSpec((1, block, H), lambda i: (i // hb, i % hb, 0))
    else:
        block = 2000

        def body(rec_r, pp_r, *rest):
            o_r = rest[-1]
            o_r[...] = tail(rec_r[...], pp_r[0] + pp_r[1], *rest[:-1])

        pspec = pl.BlockSpec((2, block, H), lambda i: (0, i, 0))

    in_specs = [pl.BlockSpec((block, H), lambda i: (i, 0)), pspec] + [_full(a) for a in full]
    return pl.pallas_call(
        body,
        grid=(R // block,),
        in_specs=in_specs,
        out_specs=pl.BlockSpec((block, H), lambda i: (i, 0)),
        out_shape=jax.ShapeDtypeStruct((R, H), _F32),
    )(rec, parts, *full)


def _sc_gather(stab, rtab, sidx, didx):
    nch = sidx.shape[0]
    E = nch * _CH
    cbase = nch // _NW
    crem = nch % _NW
    maxc = cbase + (1 if crem else 0)
    pad = jnp.zeros((1, _CH), jnp.int32)
    sidx_p = jnp.concatenate([sidx, pad], axis=0)
    didx_p = jnp.concatenate([didx, pad], axis=0)
    mesh = plsc.VectorSubcoreMesh(core_axis_name="c", subcore_axis_name="s")

    @functools.partial(
        pl.kernel,
        out_type=(jax.ShapeDtypeStruct((E, H), _F32), jax.ShapeDtypeStruct((E, H), _F32)),
        mesh=mesh,
        compiler_params=pltpu.CompilerParams(use_tc_tiling_on_sc=False),
        scratch_types=[
            pltpu.VMEM((maxc, _CH), jnp.int32),
            pltpu.VMEM((maxc, _CH), jnp.int32),
            pltpu.VMEM((2, _CH, H), _F32),
            pltpu.VMEM((2, _CH, H), _F32),
        ] + [pltpu.SemaphoreType.DMA] * 8,
    )
    def k(stab_h, rtab_h, sidx_h, didx_h, gs_h, gr_h, siv, div, sbuf, rbuf,
          gsem0, gsem1, hsem0, hsem1, wsem0, wsem1, vsem0, vsem1):
        wid = lax.axis_index("s") * _NC + lax.axis_index("c")
        cnt = cbase + (wid < crem).astype(jnp.int32)
        start = wid * cbase + jnp.minimum(wid, crem)
        pltpu.sync_copy(sidx_h.at[pl.ds(start, maxc)], siv)
        pltpu.sync_copy(didx_h.at[pl.ds(start, maxc)], div)
        gsems = (gsem0, gsem1)
        hsems = (hsem0, hsem1)
        wsems = (wsem0, wsem1)
        vsems = (vsem0, vsem1)

        def pair(ip, carry):
            for sl in (0, 1):
                jl = ip * 2 + sl

                @pl.when(jl < cnt)
                def _():
                    @pl.when(jl >= 2)
                    def _():
                        pltpu.make_async_copy(sbuf.at[sl], gs_h.at[pl.ds(0, _CH)], wsems[sl]).wait()
                        pltpu.make_async_copy(rbuf.at[sl], gr_h.at[pl.ds(0, _CH)], vsems[sl]).wait()
                    pltpu.async_copy(stab_h.at[siv.at[jl]], sbuf.at[sl], gsems[sl])
                    pltpu.async_copy(rtab_h.at[div.at[jl]], rbuf.at[sl], hsems[sl])

                jw = jl - 1
                tl = 1 - sl

                @pl.when((jw >= 0) & (jw < cnt))
                def _():
                    pltpu.make_async_copy(stab_h.at[pl.ds(0, _CH)], sbuf.at[tl], gsems[tl]).wait()
                    pltpu.make_async_copy(rtab_h.at[pl.ds(0, _CH)], rbuf.at[tl], hsems[tl]).wait()
                    j = start + jw
                    pltpu.async_copy(sbuf.at[tl], gs_h.at[pl.ds(j * _CH, _CH)], wsems[tl])
                    pltpu.async_copy(rbuf.at[tl], gr_h.at[pl.ds(j * _CH, _CH)], vsems[tl])
            return carry

        lax.fori_loop(0, cnt // 2 + 1, pair, 0)
        for sl in (0, 1):
            pltpu.make_async_copy(sbuf.at[sl], gs_h.at[pl.ds(0, _CH)], wsems[sl]).wait()
            pltpu.make_async_copy(rbuf.at[sl], gr_h.at[pl.ds(0, _CH)], vsems[sl]).wait()

    return k(stab, rtab, sidx_p, didx_p)


def _sc_scatter(ed, didx, num_rec, split):
    E = ed.shape[0]
    nch = E // _CH
    if split:
        half = num_rec // 2
        racc = half + 8
        adj = []
        for c in (0, 1):
            li = didx - c * half
            ok = (li >= 0) & (li < half)
            adj.append(jnp.where(ok, li, half))
        didx2 = jnp.stack(adj)
        nworkers = _NS
    else:
        racc = num_rec
        didx2 = jnp.stack([didx, didx])
        nworkers = _NW
    rps = racc // _NS
    cbase = nch // nworkers
    crem = nch % nworkers
    _IB = 32
    didx2 = jnp.concatenate([didx2, jnp.zeros((2, _IB, _CH), jnp.int32)], axis=1)
    zeros = jnp.zeros((racc, H), _F32)
    mesh = plsc.VectorSubcoreMesh(core_axis_name="c", subcore_axis_name="s")

    @functools.partial(
        pl.kernel,
        out_type=jax.ShapeDtypeStruct((2, racc, H), _F32),
        mesh=mesh,
        compiler_params=pltpu.CompilerParams(use_tc_tiling_on_sc=False),
        scratch_types=[
            pltpu.VMEM((2 * _IB, _CH), jnp.int32),
            pltpu.VMEM((2, _CH, H), _F32),
            pltpu.VMEM_SHARED((racc, H), _F32),
        ] + [pltpu.SemaphoreType.DMA] * 4,
    )
    def k(ed_h, didx_h, z_h, out_h, idxv, rows, accum, rsem0, rsem1, asem0, asem1):
        c = lax.axis_index("c")
        s = lax.axis_index("s")
        pltpu.sync_copy(z_h.at[pl.ds(s * rps, rps)], accum.at[pl.ds(s * rps, rps)])
        if split:
            wld = s
        else:
            wld = s * _NC + c
        cnt = cbase + (wld < crem).astype(jnp.int32)
        start = wld * cbase + jnp.minimum(wld, crem)
        plsc.subcore_barrier()
        rsems = (rsem0, rsem1)
        asems = (asem0, asem1)

        def pair(ip, carry):
            for sl in (0, 1):
                jl = ip * 2 + sl

                @pl.when(jl < cnt)
                def _():
                    @pl.when(jl % _IB == 0)
                    def _():
                        half_off = ((jl // _IB) % 2) * _IB
                        pltpu.sync_copy(didx_h.at[c, pl.ds(start + jl, _IB)],
                                        idxv.at[pl.ds(half_off, _IB)])

                    @pl.when(jl >= 2)
                    def _():
                        pltpu.make_async_copy(rows.at[sl], accum.at[pl.ds(0, _CH)], asems[sl]).wait()
                    j = start + jl
                    pltpu.async_copy(ed_h.at[pl.ds(j * _CH, _CH)], rows.at[sl], rsems[sl])

                jw = jl - 1
                tl = 1 - sl

                @pl.when((jw >= 0) & (jw < cnt))
                def _():
                    pltpu.make_async_copy(ed_h.at[pl.ds(0, _CH)], rows.at[tl], rsems[tl]).wait()
                    pltpu.async_copy(rows.at[tl], accum.at[idxv.at[jw % (2 * _IB)]], asems[tl], add=True)
            return carry

        lax.fori_loop(0, cnt // 2 + 1, pair, 0)
        for sl in (0, 1):
            pltpu.make_async_copy(rows.at[sl], accum.at[pl.ds(0, _CH)], asems[sl]).wait()
        plsc.subcore_barrier()
        pltpu.sync_copy(accum.at[pl.ds(s * rps, rps)], out_h.at[c, pl.ds(s * rps, rps)])

    return k(ed, didx2, zeros)


def _interaction(gnnp, embp, edge_index, send_rep, rec_rep, edge_feats, num_rec, split):
    em = gnnp["edge_mlp"]
    am = gnnp["aggr_mlp"]
    w1 = em["W"][0]
    w1e, w1s, w1r = w1[:H], w1[H:2 * H], w1[2 * H:]
    b1, b2 = em["b"]
    w2 = em["W"][1]
    g2, be2 = em["ln"]
    w3 = am["W"][0]
    w3r, w3a = w3[:H], w3[H:]
    b3, b4 = am["b"]
    w4 = am["W"][1]
    g4, be4 = am["ln"]

    src = edge_index[0].astype(jnp.int32)
    dst = edge_index[1].astype(jnp.int32)
    E = src.shape[0]
    sidx = src.reshape(E // _CH, _CH)
    didx = dst.reshape(E // _CH, _CH)

    sproj = _linear(send_rep, w1s)
    rproj = _linear(rec_rep, w1r)
    gs, gr = _sc_gather(sproj, rproj, sidx, didx)
    ed = _edge_dense(edge_feats, gs, gr, embp, w1e, b1, w2, b2, g2, be2)
    parts = _sc_scatter(ed, didx, num_rec, split)
    return _node_update(rec_rep, parts, w3r, w3a, b3, w4, b4, g4, be4, split)


def kernel(grid_features, g2m_features, m2g_features, mesh_static_features,
           m2m_features, params, m2m_edge_index, g2m_edge_index, m2g_edge_index):
    p = params
    grid_emb = _mlp2(grid_features, p["grid_embedder"])
    mesh_emb = _mlp2(mesh_static_features, p["mesh_embedder"])
    mesh_rep = _interaction(p["g2m_gnn"], p["g2m_embedder"], g2m_edge_index,
                            grid_emb, mesh_emb, g2m_features, 10000, False)
    grid_rep = _mlp2(grid_emb, p["encoding_grid_mlp"])
    mesh_rep = _interaction(p["processor"], p["m2m_embedder"], m2m_edge_index,
                            mesh_rep, mesh_rep, m2m_features, 10000, False)
    grid_rep = _interaction(p["m2g_gnn"], p["m2g_embedder"], m2g_edge_index,
                            mesh_rep, grid_rep, m2g_features, 50000, True)
    return _mlp2(grid_rep, p["output_map"])
